# Initial kernel scaffold; baseline (speedup 1.0000x reference)
import jax
import jax.numpy as jnp
from jax.experimental import pallas as pl


def kernel(x, edge_index, edge_attr, W1, b1, W2, b2, W3, b3, ln_gamma, ln_beta):
    raise NotImplementedError("write your pallas kernel here")


# speedup vs baseline: 5.3939x; 5.3939x over previous
import functools

import jax
import jax.numpy as jnp
from jax import lax
from jax.experimental import pallas as pl
from jax.experimental.pallas import tpu as pltpu
from jax.experimental.pallas import tpu_sc as plsc

_N = 10000
_E = 320000
_DE = 16
_DN = 128
_NC = 2
_NS = 16
_NW = _NC * _NS
_EPW = _E // _NW
_C = 80
_INNER = 25
_OUTER = 5
_BIG = _INNER * _C
_NP = 10240
_RPT = _NP // _NS


def _sc_segment_partials(col3, edge_attr):
    mesh = plsc.VectorSubcoreMesh(core_axis_name="c", subcore_axis_name="s")

    @functools.partial(
        pl.kernel,
        out_type=jax.ShapeDtypeStruct((_NC, _NP, _DE), jnp.float32),
        mesh=mesh,
        scratch_types=[
            pltpu.VMEM((_INNER, _C), jnp.int32),
            pltpu.VMEM((_BIG, _DE), jnp.float32),
            pltpu.VMEM_SHARED((_NP, _DE), jnp.float32),
            pltpu.SemaphoreType.DMA,
            pltpu.SemaphoreType.DMA,
        ],
        compiler_params=pltpu.CompilerParams(use_tc_tiling_on_sc=False),
    )
    def run(col_hbm, attr_hbm, out_hbm, idx_v, rows_v, acc_s, ld_sem, st_sem):
        cid = lax.axis_index("c")
        sid = lax.axis_index("s")
        w = cid * _NS + sid

        def zero_body(i, carry):
            rows_v[i] = jnp.zeros((_DE,), jnp.float32)
            return carry
        lax.fori_loop(0, _RPT, zero_body, None)
        pltpu.sync_copy(rows_v.at[pl.ds(0, _RPT), :],
                        acc_s.at[pl.ds(sid * _RPT, _RPT), :])
        plsc.subcore_barrier()

        for t in range(_OUTER):
            blk = w * _OUTER + t
            ld_i = pltpu.async_copy(col_hbm.at[blk], idx_v, ld_sem)
            ld_r = pltpu.async_copy(attr_hbm.at[pl.ds(blk * _BIG, _BIG), :],
                                    rows_v, ld_sem)
            ld_i.wait()
            ld_r.wait()
            descs = []
            for j in range(_INNER):
                descs.append(pltpu.async_copy(
                    rows_v.at[pl.ds(j * _C, _C), :],
                    acc_s.at[idx_v.at[j]],
                    st_sem, add=True))
            for dsc in descs:
                dsc.wait()

        plsc.subcore_barrier()
        pltpu.sync_copy(acc_s.at[pl.ds(sid * _RPT, _RPT), :],
                        out_hbm.at[cid, pl.ds(sid * _RPT, _RPT), :])

    return run(col3, edge_attr)


_BN = 1000


def _mlp_body(x_ref, p_ref, w1x_ref, w1a_ref, b1_ref, w2_ref, b2_ref,
              w3_ref, b3_ref, g_ref, be_ref, o_ref):
    xb = x_ref[...]
    agg = p_ref[0] + p_ref[1]
    h = jnp.dot(xb, w1x_ref[...], preferred_element_type=jnp.float32)
    h = h + jnp.dot(agg, w1a_ref[...], preferred_element_type=jnp.float32)
    h = jnp.maximum(h + b1_ref[...], 0.0)
    h = jnp.maximum(
        jnp.dot(h, w2_ref[...], preferred_element_type=jnp.float32) + b2_ref[...],
        0.0)
    h = jnp.dot(h, w3_ref[...], preferred_element_type=jnp.float32) + b3_ref[...]
    mu = jnp.mean(h, axis=-1, keepdims=True)
    d = h - mu
    var = jnp.mean(d * d, axis=-1, keepdims=True)
    hn = d * lax.rsqrt(var + 1e-5) * g_ref[...] + be_ref[...]
    o_ref[...] = hn + xb


def _tc_mlp(x, parts, w1x, w1a, b1, W2, b2, W3, b3, g, be):
    full = lambda i: (0, 0)
    return pl.pallas_call(
        _mlp_body,
        grid=(_N // _BN,),
        in_specs=[
            pl.BlockSpec((_BN, _DN), lambda i: (i, 0)),
            pl.BlockSpec((_NC, _BN, _DE), lambda i: (0, i, 0)),
            pl.BlockSpec((_DN, _DN), full),
            pl.BlockSpec((_DE, _DN), full),
            pl.BlockSpec((1, _DN), full),
            pl.BlockSpec((_DN, _DN), full),
            pl.BlockSpec((1, _DN), full),
            pl.BlockSpec((_DN, _DN), full),
            pl.BlockSpec((1, _DN), full),
            pl.BlockSpec((1, _DN), full),
            pl.BlockSpec((1, _DN), full),
        ],
        out_specs=pl.BlockSpec((_BN, _DN), lambda i: (i, 0)),
        out_shape=jax.ShapeDtypeStruct((_N, _DN), jnp.float32),
    )(x, parts, w1x, w1a, b1, W2, b2, W3, b3, g, be)


def kernel(x, edge_index, edge_attr, W1, b1, W2, b2, W3, b3, ln_gamma, ln_beta):
    col3 = edge_index[1].reshape(_NW * _OUTER, _INNER, _C)
    parts = _sc_segment_partials(col3, edge_attr)
    return _tc_mlp(
        x, parts, W1[:_DN], W1[_DN:], b1.reshape(1, -1), W2, b2.reshape(1, -1),
        W3, b3.reshape(1, -1), ln_gamma.reshape(1, -1), ln_beta.reshape(1, -1))


# speedup vs baseline: 5.4151x; 1.0039x over previous
import functools

import jax
import jax.numpy as jnp
from jax import lax
from jax.experimental import pallas as pl
from jax.experimental.pallas import tpu as pltpu
from jax.experimental.pallas import tpu_sc as plsc

_N = 10000
_E = 320000
_DE = 16
_DN = 128
_NC = 2
_NS = 16
_NW = _NC * _NS
_EPW = _E // _NW
_C = 80
_INNER = 25
_OUTER = 5
_BIG = _INNER * _C
_NP = 10240
_RPT = _NP // _NS


def _sc_segment_partials(ei4, edge_attr):
    mesh = plsc.VectorSubcoreMesh(core_axis_name="c", subcore_axis_name="s")

    @functools.partial(
        pl.kernel,
        out_type=jax.ShapeDtypeStruct((_NC, _NP, _DE), jnp.float32),
        mesh=mesh,
        scratch_types=[
            pltpu.VMEM((_INNER, _C), jnp.int32),
            pltpu.VMEM((_BIG, _DE), jnp.float32),
            pltpu.VMEM_SHARED((_NP, _DE), jnp.float32),
            pltpu.SemaphoreType.DMA,
            pltpu.SemaphoreType.DMA,
        ],
        compiler_params=pltpu.CompilerParams(use_tc_tiling_on_sc=False),
    )
    def run(ei_hbm, attr_hbm, out_hbm, idx_v, rows_v, acc_s, ld_sem, st_sem):
        cid = lax.axis_index("c")
        sid = lax.axis_index("s")
        w = cid * _NS + sid

        def zero_body(i, carry):
            rows_v[i] = jnp.zeros((_DE,), jnp.float32)
            return carry
        lax.fori_loop(0, _RPT, zero_body, None)
        pltpu.sync_copy(rows_v.at[pl.ds(0, _RPT), :],
                        acc_s.at[pl.ds(sid * _RPT, _RPT), :])
        plsc.subcore_barrier()

        for t in range(_OUTER):
            blk = w * _OUTER + t
            ld_i = pltpu.async_copy(ei_hbm.at[1, blk], idx_v, ld_sem)
            ld_r = pltpu.async_copy(attr_hbm.at[pl.ds(blk * _BIG, _BIG), :],
                                    rows_v, ld_sem)
            ld_i.wait()
            ld_r.wait()
            descs = []
            for j in range(_INNER):
                descs.append(pltpu.async_copy(
                    rows_v.at[pl.ds(j * _C, _C), :],
                    acc_s.at[idx_v.at[j]],
                    st_sem, add=True))
            for dsc in descs:
                dsc.wait()

        plsc.subcore_barrier()
        pltpu.sync_copy(acc_s.at[pl.ds(sid * _RPT, _RPT), :],
                        out_hbm.at[cid, pl.ds(sid * _RPT, _RPT), :])

    return run(ei4, edge_attr)


_BN = 1000


def _mlp_body(x_ref, p_ref, w1x_ref, w1a_ref, b1_ref, w2_ref, b2_ref,
              w3_ref, b3_ref, g_ref, be_ref, o_ref):
    xb = x_ref[...]
    agg = p_ref[0] + p_ref[1]
    h = jnp.dot(xb, w1x_ref[...], preferred_element_type=jnp.float32)
    h = h + jnp.dot(agg, w1a_ref[...], preferred_element_type=jnp.float32)
    h = jnp.maximum(h + b1_ref[...], 0.0)
    h = jnp.maximum(
        jnp.dot(h, w2_ref[...], preferred_element_type=jnp.float32) + b2_ref[...],
        0.0)
    h = jnp.dot(h, w3_ref[...], preferred_element_type=jnp.float32) + b3_ref[...]
    mu = jnp.mean(h, axis=-1, keepdims=True)
    d = h - mu
    var = jnp.mean(d * d, axis=-1, keepdims=True)
    hn = d * lax.rsqrt(var + 1e-5) * g_ref[...] + be_ref[...]
    o_ref[...] = hn + xb


def _tc_mlp(x, parts, w1x, w1a, b1, W2, b2, W3, b3, g, be):
    full = lambda i: (0, 0)
    return pl.pallas_call(
        _mlp_body,
        grid=(_N // _BN,),
        in_specs=[
            pl.BlockSpec((_BN, _DN), lambda i: (i, 0)),
            pl.BlockSpec((_NC, _BN, _DE), lambda i: (0, i, 0)),
            pl.BlockSpec((_DN, _DN), full),
            pl.BlockSpec((_DE, _DN), full),
            pl.BlockSpec((1, _DN), full),
            pl.BlockSpec((_DN, _DN), full),
            pl.BlockSpec((1, _DN), full),
            pl.BlockSpec((_DN, _DN), full),
            pl.BlockSpec((1, _DN), full),
            pl.BlockSpec((1, _DN), full),
            pl.BlockSpec((1, _DN), full),
        ],
        out_specs=pl.BlockSpec((_BN, _DN), lambda i: (i, 0)),
        out_shape=jax.ShapeDtypeStruct((_N, _DN), jnp.float32),
    )(x, parts, w1x, w1a, b1, W2, b2, W3, b3, g, be)


def kernel(x, edge_index, edge_attr, W1, b1, W2, b2, W3, b3, ln_gamma, ln_beta):
    ei4 = edge_index.reshape(2, _NW * _OUTER, _INNER, _C)
    parts = _sc_segment_partials(ei4, edge_attr)
    return _tc_mlp(
        x, parts, W1[:_DN], W1[_DN:], b1.reshape(1, -1), W2, b2.reshape(1, -1),
        W3, b3.reshape(1, -1), ln_gamma.reshape(1, -1), ln_beta.reshape(1, -1))


# speedup vs baseline: 7.2995x; 1.3480x over previous
import functools

import jax
import jax.numpy as jnp
from jax import lax
from jax.experimental import pallas as pl
from jax.experimental.pallas import tpu as pltpu
from jax.experimental.pallas import tpu_sc as plsc

_N = 10000
_E = 320000
_DE = 16
_DN = 128
_NC = 2
_NS = 16
_NP = 10240
_ER = _E // 128
_CRC = 1248
_CR = 208
_NCH = _CRC // _CR
_TR = _ER - _NC * _CRC


def _sc_segment_partials(col2, eat3):
    mesh = plsc.VectorSubcoreMesh(core_axis_name="c", subcore_axis_name="s")

    @functools.partial(
        pl.kernel,
        out_type=jax.ShapeDtypeStruct((_NC, _NS, _NP), jnp.float32),
        mesh=mesh,
        scratch_types=[
            pltpu.VMEM((_CR, 128), jnp.int32),
            pltpu.VMEM((_CR, 128), jnp.float32),
            pltpu.VMEM((_NP,), jnp.float32),
            pltpu.SemaphoreType.DMA,
        ],
        compiler_params=pltpu.CompilerParams(
            needs_layout_passes=False, use_tc_tiling_on_sc=False),
    )
    def run(col_hbm, val_hbm, out_hbm, idx_v, val_v, acc, ld_sem):
        cid = lax.axis_index("c")
        sid = lax.axis_index("s")

        def zero_body(i, carry):
            acc[pl.ds(i * 16, 16)] = jnp.zeros((16,), jnp.float32)
            return carry
        lax.fori_loop(0, _NP // 16, zero_body, None)

        def row_body(r, carry):
            for c in range(8):
                iv = idx_v[r, pl.ds(c * 16, 16)]
                vv = val_v[r, pl.ds(c * 16, 16)]
                plsc.addupdate_scatter(acc, [iv], vv)
            return carry

        for ch in range(_NCH):
            row0 = cid * _CRC + ch * _CR
            ld_i = pltpu.async_copy(col_hbm.at[pl.ds(row0, _CR), :], idx_v,
                                    ld_sem)
            ld_v = pltpu.async_copy(
                val_hbm.at[sid // 8, pl.ds(row0, _CR), sid % 8, :],
                val_v, ld_sem)
            ld_i.wait()
            ld_v.wait()
            lax.fori_loop(0, _CR, row_body, None)

        @pl.when(cid == _NC - 1)
        def _tail():
            ld_i = pltpu.async_copy(col_hbm.at[pl.ds(_NC * _CRC, _TR), :],
                                    idx_v.at[pl.ds(0, _TR), :], ld_sem)
            ld_v = pltpu.async_copy(
                val_hbm.at[sid // 8, pl.ds(_NC * _CRC, _TR), sid % 8, :],
                val_v.at[pl.ds(0, _TR), :], ld_sem)
            ld_i.wait()
            ld_v.wait()
            lax.fori_loop(0, _TR, row_body, None)

        pltpu.sync_copy(acc, out_hbm.at[cid, sid])

    return run(col2, eat3)


_BN = 1024


def _mlp_body(x_ref, p_ref, w1x_ref, w1a_ref, b1_ref, w2_ref, b2_ref,
              w3_ref, b3_ref, g_ref, be_ref, o_ref):
    xb = x_ref[...]
    aggt = p_ref[0] + p_ref[1]
    h = jnp.dot(xb, w1x_ref[...], preferred_element_type=jnp.float32)
    h = h + lax.dot_general(aggt, w1a_ref[...], (((0,), (0,)), ((), ())),
                            preferred_element_type=jnp.float32)
    h = jnp.maximum(h + b1_ref[...], 0.0)
    h = jnp.maximum(
        jnp.dot(h, w2_ref[...], preferred_element_type=jnp.float32) + b2_ref[...],
        0.0)
    h = jnp.dot(h, w3_ref[...], preferred_element_type=jnp.float32) + b3_ref[...]
    mu = jnp.mean(h, axis=-1, keepdims=True)
    d = h - mu
    var = jnp.mean(d * d, axis=-1, keepdims=True)
    hn = d * lax.rsqrt(var + 1e-5) * g_ref[...] + be_ref[...]
    o_ref[...] = hn + xb


def _tc_mlp(x, parts, w1x, w1a, b1, W2, b2, W3, b3, g, be):
    full = lambda i: (0, 0)
    return pl.pallas_call(
        _mlp_body,
        grid=(pl.cdiv(_N, _BN),),
        in_specs=[
            pl.BlockSpec((_BN, _DN), lambda i: (i, 0)),
            pl.BlockSpec((_NC, _NS, _BN), lambda i: (0, 0, i)),
            pl.BlockSpec((_DN, _DN), full),
            pl.BlockSpec((_DE, _DN), full),
            pl.BlockSpec((1, _DN), full),
            pl.BlockSpec((_DN, _DN), full),
            pl.BlockSpec((1, _DN), full),
            pl.BlockSpec((_DN, _DN), full),
            pl.BlockSpec((1, _DN), full),
            pl.BlockSpec((1, _DN), full),
            pl.BlockSpec((1, _DN), full),
        ],
        out_specs=pl.BlockSpec((_BN, _DN), lambda i: (i, 0)),
        out_shape=jax.ShapeDtypeStruct((_N, _DN), jnp.float32),
    )(x, parts, w1x, w1a, b1, W2, b2, W3, b3, g, be)


def kernel(x, edge_index, edge_attr, W1, b1, W2, b2, W3, b3, ln_gamma, ln_beta):
    col2 = edge_index[1].reshape(_E // 128, 128)
    eat4 = edge_attr.reshape(_E // 128, 128, 2, 8).transpose(2, 0, 3, 1)
    parts = _sc_segment_partials(col2, eat4)
    return _tc_mlp(
        x, parts, W1[:_DN], W1[_DN:], b1.reshape(1, -1), W2, b2.reshape(1, -1),
        W3, b3.reshape(1, -1), ln_gamma.reshape(1, -1), ln_beta.reshape(1, -1))


# speedup vs baseline: 9.7037x; 1.3294x over previous
import functools

import jax
import jax.numpy as jnp
from jax import lax
from jax.experimental import pallas as pl
from jax.experimental.pallas import tpu as pltpu
from jax.experimental.pallas import tpu_sc as plsc

_N = 10000
_E = 320000
_DE = 16
_DN = 128
_NC = 2
_NS = 16
_NP = 10240
_ER = _E // 128
_CRC = 1248
_CR = 208
_NCH = _CRC // _CR
_TR = _ER - _NC * _CRC


def _sc_segment_partials(col2, eat3):
    mesh = plsc.VectorSubcoreMesh(core_axis_name="c", subcore_axis_name="s")

    @functools.partial(
        pl.kernel,
        out_type=jax.ShapeDtypeStruct((_NC, _NS, _NP), jnp.float32),
        mesh=mesh,
        scratch_types=[
            pltpu.VMEM((_CR, 128), jnp.int32),
            pltpu.VMEM((_CR, 128), jnp.float32),
            pltpu.VMEM((_NP,), jnp.float32),
            pltpu.SemaphoreType.DMA,
        ],
        compiler_params=pltpu.CompilerParams(
            needs_layout_passes=False, use_tc_tiling_on_sc=False),
    )
    def run(col_hbm, val_hbm, out_hbm, idx_v, val_v, acc, ld_sem):
        cid = lax.axis_index("c")
        sid = lax.axis_index("s")

        def zero_body(i, carry):
            acc[pl.ds(i * 16, 16)] = jnp.zeros((16,), jnp.float32)
            return carry
        lax.fori_loop(0, _NP // 16, zero_body, None)

        def row_body(r):
            for c in range(8):
                iv = idx_v[r, pl.ds(c * 16, 16)]
                vv = val_v[r, pl.ds(c * 16, 16)]
                plsc.addupdate_scatter(acc, [iv], vv)

        for ch in range(_NCH):
            row0 = cid * _CRC + ch * _CR
            ld_i = pltpu.async_copy(col_hbm.at[pl.ds(row0, _CR), :], idx_v,
                                    ld_sem)
            ld_v = pltpu.async_copy(
                val_hbm.at[sid // 8, pl.ds(row0, _CR), sid % 8, :],
                val_v, ld_sem)
            ld_i.wait()
            ld_v.wait()
            plsc.parallel_loop(0, _CR, 1, unroll=2)(row_body)

        @pl.when(cid == _NC - 1)
        def _tail():
            ld_i = pltpu.async_copy(col_hbm.at[pl.ds(_NC * _CRC, _TR), :],
                                    idx_v.at[pl.ds(0, _TR), :], ld_sem)
            ld_v = pltpu.async_copy(
                val_hbm.at[sid // 8, pl.ds(_NC * _CRC, _TR), sid % 8, :],
                val_v.at[pl.ds(0, _TR), :], ld_sem)
            ld_i.wait()
            ld_v.wait()
            plsc.parallel_loop(0, _TR, 1)(row_body)

        pltpu.sync_copy(acc, out_hbm.at[cid, sid])

    return run(col2, eat3)


_BN = 1024


def _mlp_body(x_ref, p_ref, w1x_ref, w1a_ref, b1_ref, w2_ref, b2_ref,
              w3_ref, b3_ref, g_ref, be_ref, o_ref):
    xb = x_ref[...]
    aggt = p_ref[0] + p_ref[1]
    h = jnp.dot(xb, w1x_ref[...], preferred_element_type=jnp.float32)
    h = h + lax.dot_general(aggt, w1a_ref[...], (((0,), (0,)), ((), ())),
                            preferred_element_type=jnp.float32)
    h = jnp.maximum(h + b1_ref[...], 0.0)
    h = jnp.maximum(
        jnp.dot(h, w2_ref[...], preferred_element_type=jnp.float32) + b2_ref[...],
        0.0)
    h = jnp.dot(h, w3_ref[...], preferred_element_type=jnp.float32) + b3_ref[...]
    mu = jnp.mean(h, axis=-1, keepdims=True)
    d = h - mu
    var = jnp.mean(d * d, axis=-1, keepdims=True)
    hn = d * lax.rsqrt(var + 1e-5) * g_ref[...] + be_ref[...]
    o_ref[...] = hn + xb


def _tc_mlp(x, parts, w1x, w1a, b1, W2, b2, W3, b3, g, be):
    full = lambda i: (0, 0)
    return pl.pallas_call(
        _mlp_body,
        grid=(pl.cdiv(_N, _BN),),
        in_specs=[
            pl.BlockSpec((_BN, _DN), lambda i: (i, 0)),
            pl.BlockSpec((_NC, _NS, _BN), lambda i: (0, 0, i)),
            pl.BlockSpec((_DN, _DN), full),
            pl.BlockSpec((_DE, _DN), full),
            pl.BlockSpec((1, _DN), full),
            pl.BlockSpec((_DN, _DN), full),
            pl.BlockSpec((1, _DN), full),
            pl.BlockSpec((_DN, _DN), full),
            pl.BlockSpec((1, _DN), full),
            pl.BlockSpec((1, _DN), full),
            pl.BlockSpec((1, _DN), full),
        ],
        out_specs=pl.BlockSpec((_BN, _DN), lambda i: (i, 0)),
        out_shape=jax.ShapeDtypeStruct((_N, _DN), jnp.float32),
    )(x, parts, w1x, w1a, b1, W2, b2, W3, b3, g, be)


def kernel(x, edge_index, edge_attr, W1, b1, W2, b2, W3, b3, ln_gamma, ln_beta):
    col2 = edge_index[1].reshape(_E // 128, 128)
    eat4 = edge_attr.reshape(_E // 128, 128, 2, 8).transpose(2, 0, 3, 1)
    parts = _sc_segment_partials(col2, eat4)
    return _tc_mlp(
        x, parts, W1[:_DN], W1[_DN:], b1.reshape(1, -1), W2, b2.reshape(1, -1),
        W3, b3.reshape(1, -1), ln_gamma.reshape(1, -1), ln_beta.reshape(1, -1))
